# baseline (speedup 1.0000x reference)
import functools

import jax
import jax.numpy as jnp
from jax import lax
from jax.experimental import pallas as pl
from jax.experimental.pallas import tpu as pltpu
from jax.experimental.pallas import tpu_sc as plsc

D_MODEL = 64

_info = plsc.get_sparse_core_info()
_NC, _NS = _info.num_cores, _info.num_subcores
_NW = _NC * _NS

_BB = 128
_NBUF = 4


def _make_emb(batch: int, seq: int, d: int):
    assert batch == _BB * _NW
    n_chunks = seq
    dg = d // 8
    mesh = plsc.VectorSubcoreMesh(core_axis_name="c", subcore_axis_name="s")

    @functools.partial(
        pl.kernel,
        mesh=mesh,
        out_type=jax.ShapeDtypeStruct((seq, dg, _NW, 8, _BB), jnp.float32),
        compiler_params=pltpu.CompilerParams(needs_layout_passes=False),
        scratch_types=[
            pltpu.VMEM((n_chunks, _BB), jnp.int32),
            pltpu.VMEM((_NBUF, _BB * d), jnp.float32),
            pltpu.VMEM((_NBUF, dg, 8, _BB), jnp.float32),
        ]
        + [pltpu.SemaphoreType.DMA] * (2 * _NBUF),
    )
    def emb(idx_hbm, table_hbm, out_hbm, idx_v, rows_v, tiles_v, *sems):
        gsem = sems[:_NBUF]
        wsem = sems[_NBUF:]
        wid = lax.axis_index("s") * _NC + lax.axis_index("c")
        col0 = wid * _BB

        pltpu.sync_copy(idx_hbm.at[:, pl.ds(col0, _BB)], idx_v)

        lane = lax.broadcasted_iota(jnp.int32, (16,), 0)
        cgs = [(kk * 16 + lane) // 8 for kk in range(d // 16)]
        cls = [(kk * 16 + lane) % 8 for kk in range(d // 16)]

        def fire_chunk(c, b):
            def row16(v, carry):
                vec = idx_v[c, pl.ds(v * 16, 16)]
                for j in range(16):
                    t = vec[j]
                    pltpu.async_copy(
                        table_hbm.at[t],
                        rows_v.at[b, pl.ds((v * 16 + j) * d, d)],
                        gsem[b],
                    )
                return carry

            lax.fori_loop(0, _BB // 16, row16, 0)

        def transpose_chunk(b):
            def row(i, carry):
                ivec = jnp.broadcast_to(i, (16,)).astype(jnp.int32)
                for kk in range(d // 16):
                    vec = rows_v[b, pl.ds(i * d + kk * 16, 16)]
                    plsc.store_scatter(
                        tiles_v.at[b], [cgs[kk], cls[kk], ivec], vec
                    )
                return carry

            lax.fori_loop(0, _BB, row, 0)

        for b in range(_NBUF):
            fire_chunk(b, b)

        def group(g, carry):
            for b in range(_NBUF):
                c = g * _NBUF + b
                out_slice = out_hbm.at[c, :, wid]
                pltpu.make_async_copy(out_slice, tiles_v.at[b], gsem[b]).wait()

                @pl.when(c >= _NBUF)
                def _():
                    pltpu.make_async_copy(
                        tiles_v.at[b], out_hbm.at[c - _NBUF, :, wid], wsem[b]
                    ).wait()

                transpose_chunk(b)
                nc = c + _NBUF

                @pl.when(nc < n_chunks)
                def _():
                    fire_chunk(nc, b)

                pltpu.async_copy(tiles_v.at[b], out_slice, wsem[b])
            return carry

        lax.fori_loop(0, n_chunks // _NBUF, group, 0)

        for b in range(_NBUF):
            c = n_chunks - _NBUF + b
            pltpu.make_async_copy(
                tiles_v.at[b], out_hbm.at[c, :, wid], wsem[b]
            ).wait()

    return emb


def kernel(x, table):
    b, s = x.shape
    xt = x.T.astype(jnp.int32)
    out5 = _make_emb(b, s, D_MODEL)(xt, table)
    return out5.transpose(2, 4, 0, 1, 3).reshape(b, s, D_MODEL)


# speedup vs baseline: 1.2562x; 1.2562x over previous
import functools

import jax
import jax.numpy as jnp
from jax import lax
from jax.experimental import pallas as pl
from jax.experimental.pallas import tpu as pltpu
from jax.experimental.pallas import tpu_sc as plsc

D_MODEL = 64

_info = plsc.get_sparse_core_info()
_NC, _NS = _info.num_cores, _info.num_subcores
_NW = _NC * _NS

_BB = 128
_NBUF = 4


def _make_emb(batch: int, seq: int, d: int):
    assert batch == _BB * _NW
    n_chunks = seq
    dg = d // 8
    mesh = plsc.VectorSubcoreMesh(core_axis_name="c", subcore_axis_name="s")

    @functools.partial(
        pl.kernel,
        mesh=mesh,
        out_type=jax.ShapeDtypeStruct((seq, dg, _NW, 8, _BB), jnp.float32),
        compiler_params=pltpu.CompilerParams(needs_layout_passes=False),
        scratch_types=[
            pltpu.VMEM((n_chunks, _BB), jnp.int32),
            pltpu.VMEM((_NBUF, _BB * d), jnp.float32),
            pltpu.VMEM((_NBUF, dg, 8, _BB), jnp.float32),
        ]
        + [pltpu.SemaphoreType.DMA] * (2 * _NBUF),
    )
    def emb(idx_hbm, table_hbm, out_hbm, idx_v, rows_v, tiles_v, *sems):
        gsem = sems[:_NBUF]
        wsem = sems[_NBUF:]
        wid = lax.axis_index("s") * _NC + lax.axis_index("c")
        col0 = wid * _BB

        pltpu.sync_copy(idx_hbm.at[:, pl.ds(col0, _BB)], idx_v)

        lane = lax.broadcasted_iota(jnp.int32, (16,), 0)
        cgs = [(kk * 16 + lane) // 8 for kk in range(d // 16)]
        cls = [(kk * 16 + lane) % 8 for kk in range(d // 16)]

        def fire_chunk(c, b):
            def row16(v, carry):
                vec = idx_v[c, pl.ds(v * 16, 16)]
                for j in range(16):
                    t = vec[j]
                    pltpu.async_copy(
                        table_hbm.at[t],
                        rows_v.at[b, pl.ds((v * 16 + j) * d, d)],
                        gsem[b],
                    )
                return carry

            lax.fori_loop(0, _BB // 16, row16, 0)

        def transpose_chunk(b):
            @plsc.parallel_loop(0, _BB, unroll=8)
            def row(i):
                ivec = jnp.broadcast_to(i, (16,)).astype(jnp.int32)
                for kk in range(d // 16):
                    vec = rows_v[b, pl.ds(i * d + kk * 16, 16)]
                    plsc.store_scatter(
                        tiles_v.at[b], [cgs[kk], cls[kk], ivec], vec
                    )

        for b in range(_NBUF):
            fire_chunk(b, b)

        def group(g, carry):
            for b in range(_NBUF):
                c = g * _NBUF + b
                out_slice = out_hbm.at[c, :, wid]
                pltpu.make_async_copy(out_slice, tiles_v.at[b], gsem[b]).wait()

                @pl.when(c >= _NBUF)
                def _():
                    pltpu.make_async_copy(
                        tiles_v.at[b], out_hbm.at[c - _NBUF, :, wid], wsem[b]
                    ).wait()

                transpose_chunk(b)
                nc = c + _NBUF

                @pl.when(nc < n_chunks)
                def _():
                    fire_chunk(nc, b)

                pltpu.async_copy(tiles_v.at[b], out_slice, wsem[b])
            return carry

        lax.fori_loop(0, n_chunks // _NBUF, group, 0)

        for b in range(_NBUF):
            c = n_chunks - _NBUF + b
            pltpu.make_async_copy(
                tiles_v.at[b], out_hbm.at[c, :, wid], wsem[b]
            ).wait()

    return emb


def kernel(x, table):
    b, s = x.shape
    xt = x.T.astype(jnp.int32)
    out5 = _make_emb(b, s, D_MODEL)(xt, table)
    return out5.transpose(2, 4, 0, 1, 3).reshape(b, s, D_MODEL)


# speedup vs baseline: 1.7075x; 1.3593x over previous
import functools

import jax
import jax.numpy as jnp
from jax import lax
from jax.experimental import pallas as pl
from jax.experimental.pallas import tpu as pltpu
from jax.experimental.pallas import tpu_sc as plsc

D_MODEL = 64

_info = plsc.get_sparse_core_info()
_NC, _NS = _info.num_cores, _info.num_subcores
_NW = _NC * _NS

_CHUNK = 128
_NBUF = 4


def _make_emb(n_rows: int, d: int):
    rows_per_w = n_rows // _NW
    n_chunks = rows_per_w // _CHUNK
    assert n_chunks % _NBUF == 0
    mesh = plsc.VectorSubcoreMesh(core_axis_name="c", subcore_axis_name="s")

    @functools.partial(
        pl.kernel,
        mesh=mesh,
        out_type=jax.ShapeDtypeStruct((n_rows, d), jnp.float32),
        compiler_params=pltpu.CompilerParams(needs_layout_passes=False),
        scratch_types=[
            pltpu.VMEM((n_chunks, _CHUNK), jnp.int32),
            pltpu.VMEM((_NBUF, _CHUNK, d), jnp.float32),
        ]
        + [pltpu.SemaphoreType.DMA] * (2 * _NBUF),
    )
    def emb(idx_hbm, table_hbm, out_hbm, idx_v, rows_v, *sems):
        gsem = sems[:_NBUF]
        wsem = sems[_NBUF:]
        wid = lax.axis_index("s") * _NC + lax.axis_index("c")
        chunk_base = wid * n_chunks
        row_base = chunk_base * _CHUNK

        pltpu.sync_copy(idx_hbm.at[pl.ds(chunk_base, n_chunks)], idx_v)

        def fire_chunk(c, b):
            @plsc.parallel_loop(0, _CHUNK // 16, unroll=2)
            def row16(v):
                vec = idx_v[c, pl.ds(v * 16, 16)]
                for j in range(16):
                    t = vec[j]
                    pltpu.async_copy(
                        table_hbm.at[t], rows_v.at[b, v * 16 + j], gsem[b]
                    )

        for b in range(_NBUF):
            fire_chunk(b, b)

        def group(g, carry):
            for b in range(_NBUF):
                c = g * _NBUF + b
                row_off = row_base + c * _CHUNK
                out_slice = out_hbm.at[pl.ds(row_off, _CHUNK)]
                pltpu.make_async_copy(out_slice, rows_v.at[b], gsem[b]).wait()
                pltpu.async_copy(rows_v.at[b], out_slice, wsem[b])
                nc = c + _NBUF

                @pl.when(nc < n_chunks)
                def _():
                    pltpu.make_async_copy(
                        rows_v.at[b], out_slice, wsem[b]
                    ).wait()
                    fire_chunk(nc, b)

            return carry

        lax.fori_loop(0, n_chunks // _NBUF, group, 0)

        for b in range(_NBUF):
            c = n_chunks - _NBUF + b
            pltpu.make_async_copy(
                rows_v.at[b],
                out_hbm.at[pl.ds(row_base + c * _CHUNK, _CHUNK)],
                wsem[b],
            ).wait()

    return emb


def kernel(x, table):
    b, s = x.shape
    n = b * s
    idx2d = x.reshape(n // _CHUNK, _CHUNK).astype(jnp.int32)
    out = _make_emb(n, D_MODEL)(idx2d, table)
    return out.reshape(b, s, D_MODEL)
